# Initial kernel scaffold; baseline (speedup 1.0000x reference)
import jax
import jax.numpy as jnp
from jax.experimental import pallas as pl


def kernel(x, edge_index, W0, W1, b1, W2, b2, W3, b3):
    raise NotImplementedError("write your pallas kernel here")


# speedup vs baseline: 11.6058x; 11.6058x over previous
import functools

import jax
import jax.numpy as jnp
from jax import lax
from jax.experimental import pallas as pl
from jax.experimental.pallas import tpu as pltpu
from jax.experimental.pallas import tpu_sc as plsc

_N = 10000
_D = 512
_NT = 4
_CW = 128
_BN = 1000
_PAD_ROWS = 256


def _mesh():
    return plsc.VectorSubcoreMesh(
        core_axis_name="c", subcore_axis_name="s", num_cores=2, num_subcores=16
    )


def _tile_rows(s_static):
    if s_static < 15:
        return s_static * 640, 640
    return 9600, 400


def _zero_slices(acc_ref, zbuf, s):
    for s_static in range(16):
        @pl.when(s == s_static)
        def _():
            off, cnt = _tile_rows(s_static)

            def zb(i, _):
                pltpu.sync_copy(zbuf, acc_ref.at[pl.ds(off + i * 16, 16)])
                return ()

            lax.fori_loop(0, cnt // 16, zb, ())


def _writeout(acc_ref, out_ref, s):
    for s_static in range(16):
        @pl.when(s == s_static)
        def _():
            off, cnt = _tile_rows(s_static)
            pltpu.sync_copy(
                acc_ref.at[pl.ds(off, cnt)], out_ref.at[pl.ds(off, cnt)]
            )


def _deg_call(dst2d):
    rows = dst2d.shape[0]
    rpt = rows // 32
    ones = jnp.ones((64, 128), jnp.float32)
    zeros = jnp.zeros((640, 128), jnp.float32)

    @functools.partial(
        pl.kernel,
        out_type=jax.ShapeDtypeStruct((2, _N, 128), jnp.float32),
        mesh=_mesh(),
        scratch_types=(
            pltpu.VMEM_SHARED((_N + _PAD_ROWS, 128), jnp.float32),
            pltpu.VMEM((rpt, 64), jnp.int32),
            pltpu.VMEM((64, 128), jnp.float32),
        ),
    )
    def k(dst_hbm, ones_hbm, zeros_hbm, out_hbm, dacc, dst_v, ones_v):
        c = lax.axis_index("c")
        s = lax.axis_index("s")

        pltpu.sync_copy(ones_hbm, ones_v)
        for s_static in range(16):
            @pl.when(s == s_static)
            def _():
                off, cnt = _tile_rows(s_static)
                pltpu.sync_copy(
                    zeros_hbm.at[pl.ds(0, cnt)], dacc.at[pl.ds(off, cnt)]
                )
        plsc.subcore_barrier()

        base = (c * 16 + s) * rpt
        pltpu.sync_copy(dst_hbm.at[pl.ds(base, rpt)], dst_v)

        def body(j, _):
            pltpu.sync_copy(ones_v, dacc.at[dst_v.at[j]], add=True)
            return ()

        lax.fori_loop(0, rpt, body, ())
        plsc.subcore_barrier()
        _writeout(dacc, out_hbm.at[c], s)

    return k(dst2d, ones, zeros)


_BLK = 32


def _conv_call(tab, srcg, dst2d):
    rows = srcg.shape[1]
    rpt = rows // 16
    nblk = rpt // _BLK

    @functools.partial(
        pl.kernel,
        out_type=jax.ShapeDtypeStruct((_NT, _N, _CW), jnp.float32),
        mesh=_mesh(),
        scratch_types=(
            pltpu.VMEM_SHARED((_N + _PAD_ROWS, _CW), jnp.float32),
            pltpu.VMEM((_BLK, 64), jnp.int32),
            pltpu.VMEM((_BLK, 64), jnp.int32),
            pltpu.VMEM((64, _CW), jnp.float32),
            pltpu.VMEM((64, _CW), jnp.float32),
            pltpu.VMEM((16, _CW), jnp.float32),
            pltpu.SemaphoreType.DMA,
            pltpu.SemaphoreType.DMA,
        ),
    )
    def k(tab_hbm, srcg_hbm, dst_hbm, out_hbm,
          accum, src_v, dst_v, rows0, rows1, zbuf, sem0, sem1):
        c = lax.axis_index("c")
        s = lax.axis_index("s")
        base = s * rpt

        def fill_zero(i, _):
            for kk in range(_CW // 16):
                zbuf[i, pl.ds(kk * 16, 16)] = jnp.zeros((16,), jnp.float32)
            return ()

        lax.fori_loop(0, 16, fill_zero, ())

        for p in range(2):
            tbl = 2 * p + c
            _zero_slices(accum, zbuf, s)
            plsc.subcore_barrier()

            for bi in range(nblk):
                boff = base + bi * _BLK
                pltpu.sync_copy(srcg_hbm.at[tbl, pl.ds(boff, _BLK)], src_v)
                pltpu.sync_copy(dst_hbm.at[pl.ds(boff, _BLK)], dst_v)

                pltpu.async_copy(tab_hbm.at[src_v.at[0]], rows0, sem0)

                def body(jj, _):
                    j0 = 2 * jj
                    j1 = 2 * jj + 1
                    pltpu.async_copy(tab_hbm.at[src_v.at[j1]], rows1, sem1)
                    pltpu.make_async_copy(
                        tab_hbm.at[src_v.at[j0]], rows0, sem0
                    ).wait()
                    pltpu.sync_copy(rows0, accum.at[dst_v.at[j0]], add=True)

                    @pl.when(jj < _BLK // 2 - 1)
                    def _():
                        pltpu.async_copy(
                            tab_hbm.at[src_v.at[j0 + 2]], rows0, sem0
                        )

                    pltpu.make_async_copy(
                        tab_hbm.at[src_v.at[j1]], rows1, sem1
                    ).wait()
                    pltpu.sync_copy(rows1, accum.at[dst_v.at[j1]], add=True)
                    return ()

                lax.fori_loop(0, _BLK // 2, body, ())

            plsc.subcore_barrier()
            _writeout(accum, out_hbm.at[tbl], s)
            if p == 0:
                plsc.subcore_barrier()

    return k(tab, srcg, dst2d)


def _wprep_call(w0, w1):
    def k(w0_ref, w1_ref, out_ref):
        out_ref[:, :] = jnp.dot(
            w0_ref[:, :], w1_ref[:, :], preferred_element_type=jnp.float32
        )

    return pl.pallas_call(
        k, out_shape=jax.ShapeDtypeStruct((w0.shape[0], _D), jnp.float32)
    )(w0, w1)


def _dinv_of(degp_ref):
    deg = degp_ref[0, :, 0:1] + degp_ref[1, :, 0:1] + 1.0
    return lax.rsqrt(deg)


def _m1_call(x, w01, degp):
    n, kdim = x.shape

    def k(x_ref, w_ref, degp_ref, out_ref):
        dinv = _dinv_of(degp_ref)
        g = jnp.dot(x_ref[:, :], w_ref[:, :], preferred_element_type=jnp.float32)
        gs = g * dinv
        for cc in range(_NT):
            out_ref[cc, :, :] = gs[:, cc * _CW:(cc + 1) * _CW]

    return pl.pallas_call(
        k,
        grid=(n // _BN,),
        in_specs=[
            pl.BlockSpec((_BN, kdim), lambda i: (i, 0)),
            pl.BlockSpec((kdim, _D), lambda i: (0, 0)),
            pl.BlockSpec((2, _BN, 128), lambda i: (0, i, 0)),
        ],
        out_specs=pl.BlockSpec((_NT, _BN, _CW), lambda i: (0, i, 0)),
        out_shape=jax.ShapeDtypeStruct((_NT, n, _CW), jnp.float32),
    )(x, w01, degp)


def _mmid_call(acc, gs, degp, b4, w4):
    def k(acc_ref, gs_ref, degp_ref, b_ref, w_ref, out_ref):
        dinv = _dinv_of(degp_ref)
        o = jnp.zeros((_BN, _D), jnp.float32)
        for cc in range(_NT):
            a_c = jnp.maximum(
                dinv * (acc_ref[cc] + gs_ref[cc]) + b_ref[cc, :][None, :], 0.0
            )
            o = o + jnp.dot(a_c, w_ref[cc], preferred_element_type=jnp.float32)
        o = o * dinv
        for cc in range(_NT):
            out_ref[cc, :, :] = o[:, cc * _CW:(cc + 1) * _CW]

    return pl.pallas_call(
        k,
        grid=(_N // _BN,),
        in_specs=[
            pl.BlockSpec((_NT, _BN, _CW), lambda i: (0, i, 0)),
            pl.BlockSpec((_NT, _BN, _CW), lambda i: (0, i, 0)),
            pl.BlockSpec((2, _BN, 128), lambda i: (0, i, 0)),
            pl.BlockSpec((_NT, _CW), lambda i: (0, 0)),
            pl.BlockSpec((_NT, _CW, _D), lambda i: (0, 0, 0)),
        ],
        out_specs=pl.BlockSpec((_NT, _BN, _CW), lambda i: (0, i, 0)),
        out_shape=jax.ShapeDtypeStruct((_NT, _N, _CW), jnp.float32),
    )(acc, gs, degp, b4, w4)


def _final_call(acc, gs, degp, b4):
    def k(acc_ref, gs_ref, degp_ref, b_ref, out_ref):
        dinv = _dinv_of(degp_ref)
        for cc in range(_NT):
            out_ref[:, cc * _CW:(cc + 1) * _CW] = jnp.maximum(
                dinv * (acc_ref[cc] + gs_ref[cc]) + b_ref[cc, :][None, :], 0.0
            )

    return pl.pallas_call(
        k,
        grid=(_N // _BN,),
        in_specs=[
            pl.BlockSpec((_NT, _BN, _CW), lambda i: (0, i, 0)),
            pl.BlockSpec((_NT, _BN, _CW), lambda i: (0, i, 0)),
            pl.BlockSpec((2, _BN, 128), lambda i: (0, i, 0)),
            pl.BlockSpec((_NT, _CW), lambda i: (0, 0)),
        ],
        out_specs=pl.BlockSpec((_BN, _D), lambda i: (i, 0)),
        out_shape=jax.ShapeDtypeStruct((_N, _D), jnp.float32),
    )(acc, gs, degp, b4)


def kernel(x, edge_index, W0, W1, b1, W2, b2, W3, b3):
    e = edge_index.shape[1]
    src = edge_index[0].astype(jnp.int32)
    dst = edge_index[1].astype(jnp.int32)

    rows = -(-e // (512 * 64)) * 512
    npad = rows * 64 - e
    pad_ar = jnp.arange(npad, dtype=jnp.int32)
    src_p = jnp.concatenate([src, pad_ar % _N]).reshape(rows, 64)
    dst_p = jnp.concatenate([dst, _N + pad_ar % _PAD_ROWS]).reshape(rows, 64)
    srcg = src_p[None, :, :] + (_N * jnp.arange(_NT, dtype=jnp.int32))[:, None, None]

    degp = _deg_call(dst_p)
    w01 = _wprep_call(W0, W1)
    b1_4 = b1.reshape(_NT, _CW)
    b2_4 = b2.reshape(_NT, _CW)
    b3_4 = b3.reshape(_NT, _CW)
    w2_4 = W2.reshape(_NT, _CW, _D)
    w3_4 = W3.reshape(_NT, _CW, _D)

    gs1 = _m1_call(x, w01, degp)
    acc1 = _conv_call(gs1.reshape(_NT * _N, _CW), srcg, dst_p)
    gs2 = _mmid_call(acc1, gs1, degp, b1_4, w2_4)
    acc2 = _conv_call(gs2.reshape(_NT * _N, _CW), srcg, dst_p)
    gs3 = _mmid_call(acc2, gs2, degp, b2_4, w3_4)
    acc3 = _conv_call(gs3.reshape(_NT * _N, _CW), srcg, dst_p)
    h = _final_call(acc3, gs3, degp, b3_4)
    return h.reshape(100, -1, _D)


# speedup vs baseline: 13.3084x; 1.1467x over previous
import functools

import jax
import jax.numpy as jnp
from jax import lax
from jax.experimental import pallas as pl
from jax.experimental.pallas import tpu as pltpu
from jax.experimental.pallas import tpu_sc as plsc

_N = 10000
_D = 512
_NT = 4
_CW = 128
_BN = 1000
_PAD_ROWS = 256


def _mesh():
    return plsc.VectorSubcoreMesh(
        core_axis_name="c", subcore_axis_name="s", num_cores=2, num_subcores=16
    )


def _tile_rows(s_static):
    if s_static < 15:
        return s_static * 640, 640
    return 9600, 400


def _zero_slices(acc_ref, zbuf, s):
    for s_static in range(16):
        @pl.when(s == s_static)
        def _():
            off, cnt = _tile_rows(s_static)

            def zb(i, _):
                pltpu.sync_copy(zbuf, acc_ref.at[pl.ds(off + i * 16, 16)])
                return ()

            lax.fori_loop(0, cnt // 16, zb, ())


def _writeout(acc_ref, out_ref, s):
    for s_static in range(16):
        @pl.when(s == s_static)
        def _():
            off, cnt = _tile_rows(s_static)
            pltpu.sync_copy(
                acc_ref.at[pl.ds(off, cnt)], out_ref.at[pl.ds(off, cnt)]
            )


def _deg_call(dst2d):
    rows = dst2d.shape[0]
    rpt = rows // 32
    ones = jnp.ones((64, 128), jnp.float32)
    zeros = jnp.zeros((640, 128), jnp.float32)

    @functools.partial(
        pl.kernel,
        out_type=jax.ShapeDtypeStruct((2, _N, 128), jnp.float32),
        mesh=_mesh(),
        scratch_types=(
            pltpu.VMEM_SHARED((_N + _PAD_ROWS, 128), jnp.float32),
            pltpu.VMEM((rpt, 64), jnp.int32),
            pltpu.VMEM((64, 128), jnp.float32),
        ),
    )
    def k(dst_hbm, ones_hbm, zeros_hbm, out_hbm, dacc, dst_v, ones_v):
        c = lax.axis_index("c")
        s = lax.axis_index("s")

        pltpu.sync_copy(ones_hbm, ones_v)
        for s_static in range(16):
            @pl.when(s == s_static)
            def _():
                off, cnt = _tile_rows(s_static)
                pltpu.sync_copy(
                    zeros_hbm.at[pl.ds(0, cnt)], dacc.at[pl.ds(off, cnt)]
                )
        plsc.subcore_barrier()

        base = (c * 16 + s) * rpt
        pltpu.sync_copy(dst_hbm.at[pl.ds(base, rpt)], dst_v)

        def body(j, _):
            pltpu.sync_copy(ones_v, dacc.at[dst_v.at[j]], add=True)
            return ()

        lax.fori_loop(0, rpt, body, ())
        plsc.subcore_barrier()
        _writeout(dacc, out_hbm.at[c], s)

    return k(dst2d, ones, zeros)


_BLK = 32


def _conv_call(tab, srcg, dst2d):
    rows = srcg.shape[1]
    rpt = rows // 16
    nblk = rpt // _BLK

    @functools.partial(
        pl.kernel,
        out_type=jax.ShapeDtypeStruct((_NT, _N, _CW), jnp.float32),
        mesh=_mesh(),
        scratch_types=(
            pltpu.VMEM_SHARED((_N + _PAD_ROWS, _CW), jnp.float32),
            pltpu.VMEM((_BLK, 64), jnp.int32),
            pltpu.VMEM((_BLK, 64), jnp.int32),
            pltpu.VMEM((64, _CW), jnp.float32),
            pltpu.VMEM((64, _CW), jnp.float32),
            pltpu.VMEM((64, _CW), jnp.float32),
            pltpu.VMEM((16, _CW), jnp.float32),
            pltpu.SemaphoreType.DMA,
            pltpu.SemaphoreType.DMA,
            pltpu.SemaphoreType.DMA,
            pltpu.SemaphoreType.DMA,
            pltpu.SemaphoreType.DMA,
            pltpu.SemaphoreType.DMA,
        ),
    )
    def k(tab_hbm, srcg_hbm, dst_hbm, out_hbm,
          accum, src_v, dst_v, ra, rb, rc, zbuf, ga, gb, gc, sa, sb, sc):
        c = lax.axis_index("c")
        s = lax.axis_index("s")
        base = s * rpt
        bufs = (ra, rb, rc)
        gsems = (ga, gb, gc)
        ssems = (sa, sb, sc)

        def g_start(j, q):
            pltpu.async_copy(tab_hbm.at[src_v.at[j]], bufs[q], gsems[q])

        def g_wait(j, q):
            pltpu.make_async_copy(
                tab_hbm.at[src_v.at[j]], bufs[q], gsems[q]
            ).wait()

        def s_start(j, q):
            pltpu.async_copy(
                bufs[q], accum.at[dst_v.at[j]], ssems[q], add=True
            )

        def s_wait(j, q):
            pltpu.make_async_copy(
                bufs[q], accum.at[dst_v.at[j]], ssems[q]
            ).wait()

        def fill_zero(i, _):
            for kk in range(_CW // 16):
                zbuf[i, pl.ds(kk * 16, 16)] = jnp.zeros((16,), jnp.float32)
            return ()

        lax.fori_loop(0, 16, fill_zero, ())

        for p in range(2):
            tbl = 2 * p + c
            _zero_slices(accum, zbuf, s)
            plsc.subcore_barrier()

            for bi in range(nblk):
                boff = base + bi * _BLK
                pltpu.sync_copy(srcg_hbm.at[tbl, pl.ds(boff, _BLK)], src_v)
                pltpu.sync_copy(dst_hbm.at[pl.ds(boff, _BLK)], dst_v)

                g_start(0, 0)
                g_start(1, 1)
                g_wait(0, 0)
                s_start(0, 0)
                g_start(2, 2)
                g_wait(1, 1)
                s_start(1, 1)
                s_wait(0, 0)
                g_start(3, 0)
                last = _BLK - 1
                ntrip = (_BLK - 2) // 3

                def body(t, _):
                    j2 = 3 * t + 2
                    g_wait(j2, 2)
                    s_start(j2, 2)
                    s_wait(j2 - 1, 1)
                    g_start(j2 + 2, 1)
                    g_wait(j2 + 1, 0)
                    s_start(j2 + 1, 0)

                    @pl.when(j2 + 3 <= last)
                    def _():
                        s_wait(j2, 2)
                        g_start(j2 + 3, 2)

                    g_wait(j2 + 2, 1)
                    s_start(j2 + 2, 1)

                    @pl.when(j2 + 4 <= last)
                    def _():
                        s_wait(j2 + 1, 0)
                        g_start(j2 + 4, 0)

                    return ()

                lax.fori_loop(0, ntrip, body, ())
                s_wait(last - 2, 2)
                s_wait(last - 1, 0)
                s_wait(last, 1)

            plsc.subcore_barrier()
            _writeout(accum, out_hbm.at[tbl], s)
            if p == 0:
                plsc.subcore_barrier()

    return k(tab, srcg, dst2d)


def _wprep_call(w0, w1):
    def k(w0_ref, w1_ref, out_ref):
        out_ref[:, :] = jnp.dot(
            w0_ref[:, :], w1_ref[:, :], preferred_element_type=jnp.float32
        )

    return pl.pallas_call(
        k, out_shape=jax.ShapeDtypeStruct((w0.shape[0], _D), jnp.float32)
    )(w0, w1)


def _dinv_of(degp_ref):
    deg = degp_ref[0, :, 0:1] + degp_ref[1, :, 0:1] + 1.0
    return lax.rsqrt(deg)


def _m1_call(x, w01, degp):
    n, kdim = x.shape

    def k(x_ref, w_ref, degp_ref, out_ref):
        dinv = _dinv_of(degp_ref)
        g = jnp.dot(x_ref[:, :], w_ref[:, :], preferred_element_type=jnp.float32)
        gs = g * dinv
        for cc in range(_NT):
            out_ref[cc, :, :] = gs[:, cc * _CW:(cc + 1) * _CW]

    return pl.pallas_call(
        k,
        grid=(n // _BN,),
        in_specs=[
            pl.BlockSpec((_BN, kdim), lambda i: (i, 0)),
            pl.BlockSpec((kdim, _D), lambda i: (0, 0)),
            pl.BlockSpec((2, _BN, 128), lambda i: (0, i, 0)),
        ],
        out_specs=pl.BlockSpec((_NT, _BN, _CW), lambda i: (0, i, 0)),
        out_shape=jax.ShapeDtypeStruct((_NT, n, _CW), jnp.float32),
    )(x, w01, degp)


def _mmid_call(acc, gs, degp, b4, w4):
    def k(acc_ref, gs_ref, degp_ref, b_ref, w_ref, out_ref):
        dinv = _dinv_of(degp_ref)
        o = jnp.zeros((_BN, _D), jnp.float32)
        for cc in range(_NT):
            a_c = jnp.maximum(
                dinv * (acc_ref[cc] + gs_ref[cc]) + b_ref[cc, :][None, :], 0.0
            )
            o = o + jnp.dot(a_c, w_ref[cc], preferred_element_type=jnp.float32)
        o = o * dinv
        for cc in range(_NT):
            out_ref[cc, :, :] = o[:, cc * _CW:(cc + 1) * _CW]

    return pl.pallas_call(
        k,
        grid=(_N // _BN,),
        in_specs=[
            pl.BlockSpec((_NT, _BN, _CW), lambda i: (0, i, 0)),
            pl.BlockSpec((_NT, _BN, _CW), lambda i: (0, i, 0)),
            pl.BlockSpec((2, _BN, 128), lambda i: (0, i, 0)),
            pl.BlockSpec((_NT, _CW), lambda i: (0, 0)),
            pl.BlockSpec((_NT, _CW, _D), lambda i: (0, 0, 0)),
        ],
        out_specs=pl.BlockSpec((_NT, _BN, _CW), lambda i: (0, i, 0)),
        out_shape=jax.ShapeDtypeStruct((_NT, _N, _CW), jnp.float32),
    )(acc, gs, degp, b4, w4)


def _final_call(acc, gs, degp, b4):
    def k(acc_ref, gs_ref, degp_ref, b_ref, out_ref):
        dinv = _dinv_of(degp_ref)
        for cc in range(_NT):
            out_ref[:, cc * _CW:(cc + 1) * _CW] = jnp.maximum(
                dinv * (acc_ref[cc] + gs_ref[cc]) + b_ref[cc, :][None, :], 0.0
            )

    return pl.pallas_call(
        k,
        grid=(_N // _BN,),
        in_specs=[
            pl.BlockSpec((_NT, _BN, _CW), lambda i: (0, i, 0)),
            pl.BlockSpec((_NT, _BN, _CW), lambda i: (0, i, 0)),
            pl.BlockSpec((2, _BN, 128), lambda i: (0, i, 0)),
            pl.BlockSpec((_NT, _CW), lambda i: (0, 0)),
        ],
        out_specs=pl.BlockSpec((_BN, _D), lambda i: (i, 0)),
        out_shape=jax.ShapeDtypeStruct((_N, _D), jnp.float32),
    )(acc, gs, degp, b4)


def kernel(x, edge_index, W0, W1, b1, W2, b2, W3, b3):
    e = edge_index.shape[1]
    src = edge_index[0].astype(jnp.int32)
    dst = edge_index[1].astype(jnp.int32)

    rows = -(-e // (512 * 64)) * 512
    npad = rows * 64 - e
    pad_ar = jnp.arange(npad, dtype=jnp.int32)
    src_p = jnp.concatenate([src, pad_ar % _N]).reshape(rows, 64)
    dst_p = jnp.concatenate([dst, _N + pad_ar % _PAD_ROWS]).reshape(rows, 64)
    srcg = src_p[None, :, :] + (_N * jnp.arange(_NT, dtype=jnp.int32))[:, None, None]

    degp = _deg_call(dst_p)
    w01 = _wprep_call(W0, W1)
    b1_4 = b1.reshape(_NT, _CW)
    b2_4 = b2.reshape(_NT, _CW)
    b3_4 = b3.reshape(_NT, _CW)
    w2_4 = W2.reshape(_NT, _CW, _D)
    w3_4 = W3.reshape(_NT, _CW, _D)

    gs1 = _m1_call(x, w01, degp)
    acc1 = _conv_call(gs1.reshape(_NT * _N, _CW), srcg, dst_p)
    gs2 = _mmid_call(acc1, gs1, degp, b1_4, w2_4)
    acc2 = _conv_call(gs2.reshape(_NT * _N, _CW), srcg, dst_p)
    gs3 = _mmid_call(acc2, gs2, degp, b2_4, w3_4)
    acc3 = _conv_call(gs3.reshape(_NT * _N, _CW), srcg, dst_p)
    h = _final_call(acc3, gs3, degp, b3_4)
    return h.reshape(100, -1, _D)


# speedup vs baseline: 13.6260x; 1.0239x over previous
import functools

import jax
import jax.numpy as jnp
from jax import lax
from jax.experimental import pallas as pl
from jax.experimental.pallas import tpu as pltpu
from jax.experimental.pallas import tpu_sc as plsc

_N = 10000
_D = 512
_NT = 4
_CW = 128
_BN = 1000
_PAD_ROWS = 16


def _mesh():
    return plsc.VectorSubcoreMesh(
        core_axis_name="c", subcore_axis_name="s", num_cores=2, num_subcores=16
    )


def _tile_rows(s_static):
    if s_static < 15:
        return s_static * 640, 640
    return 9600, 400


def _zero_slices(acc_ref, zbuf, s):
    for s_static in range(16):
        @pl.when(s == s_static)
        def _():
            off, cnt = _tile_rows(s_static)

            def zb(i, _):
                pltpu.sync_copy(zbuf, acc_ref.at[pl.ds(off + i * 8, 8)])
                return ()

            lax.fori_loop(0, cnt // 8, zb, ())


def _writeout(acc_ref, out_ref, s):
    for s_static in range(16):
        @pl.when(s == s_static)
        def _():
            off, cnt = _tile_rows(s_static)
            pltpu.sync_copy(
                acc_ref.at[pl.ds(off, cnt)], out_ref.at[pl.ds(off, cnt)]
            )


def _deg_call(dst2d):
    rows = dst2d.shape[0]
    rpt = rows // 32
    ones = jnp.ones((64, 128), jnp.float32)
    zeros = jnp.zeros((640, 128), jnp.float32)

    @functools.partial(
        pl.kernel,
        out_type=jax.ShapeDtypeStruct((2, _N, 128), jnp.float32),
        mesh=_mesh(),
        scratch_types=(
            pltpu.VMEM_SHARED((_N + _PAD_ROWS, 128), jnp.float32),
            pltpu.VMEM((rpt, 64), jnp.int32),
            pltpu.VMEM((64, 128), jnp.float32),
        ),
    )
    def k(dst_hbm, ones_hbm, zeros_hbm, out_hbm, dacc, dst_v, ones_v):
        c = lax.axis_index("c")
        s = lax.axis_index("s")

        pltpu.sync_copy(ones_hbm, ones_v)
        for s_static in range(16):
            @pl.when(s == s_static)
            def _():
                off, cnt = _tile_rows(s_static)
                pltpu.sync_copy(
                    zeros_hbm.at[pl.ds(0, cnt)], dacc.at[pl.ds(off, cnt)]
                )
        plsc.subcore_barrier()

        base = (c * 16 + s) * rpt
        pltpu.sync_copy(dst_hbm.at[pl.ds(base, rpt)], dst_v)

        def body(j, _):
            pltpu.sync_copy(ones_v, dacc.at[dst_v.at[j]], add=True)
            return ()

        lax.fori_loop(0, rpt, body, ())
        plsc.subcore_barrier()
        _writeout(dacc, out_hbm.at[c], s)

    return k(dst2d, ones, zeros)


_BLK = 32


def _conv_call(tab, srcg, dst2d):
    rows = srcg.shape[1]
    rpt = rows // 16
    nblk = rpt // _BLK

    @functools.partial(
        pl.kernel,
        out_type=jax.ShapeDtypeStruct((_NT, _N, _CW), jnp.float32),
        mesh=_mesh(),
        scratch_types=(
            pltpu.VMEM_SHARED((_N + _PAD_ROWS, _CW), jnp.float32),
            pltpu.VMEM((2, _BLK, 64), jnp.int32),
            pltpu.VMEM((2, _BLK, 64), jnp.int32),
            pltpu.VMEM((64, _CW), jnp.float32),
            pltpu.VMEM((64, _CW), jnp.float32),
            pltpu.VMEM((64, _CW), jnp.float32),
            pltpu.VMEM((8, _CW), jnp.float32),
            pltpu.SemaphoreType.DMA,
            pltpu.SemaphoreType.DMA,
            pltpu.SemaphoreType.DMA,
            pltpu.SemaphoreType.DMA,
            pltpu.SemaphoreType.DMA,
            pltpu.SemaphoreType.DMA,
            pltpu.SemaphoreType.DMA,
            pltpu.SemaphoreType.DMA,
        ),
    )
    def k(tab_hbm, srcg_hbm, dst_hbm, out_hbm,
          accum, srcd_v, dstd_v, ra, rb, rc, zbuf,
          ga, gb, gc, sa, sb, sc, ia, ib):
        c = lax.axis_index("c")
        s = lax.axis_index("s")
        base = s * rpt
        bufs = (ra, rb, rc)
        gsems = (ga, gb, gc)
        ssems = (sa, sb, sc)

        def idx_refs(par):
            return srcd_v.at[par], dstd_v.at[par]

        def idx_start(bi, par, tbl):
            boff = base + bi * _BLK
            isem = (ia, ib)[par]
            sv, dv = idx_refs(par)
            pltpu.async_copy(srcg_hbm.at[tbl, pl.ds(boff, _BLK)], sv, isem)
            pltpu.async_copy(dst_hbm.at[pl.ds(boff, _BLK)], dv, isem)

        def idx_wait(bi, par, tbl):
            boff = base + bi * _BLK
            isem = (ia, ib)[par]
            sv, dv = idx_refs(par)
            pltpu.make_async_copy(
                srcg_hbm.at[tbl, pl.ds(boff, _BLK)], sv, isem
            ).wait()
            pltpu.make_async_copy(
                dst_hbm.at[pl.ds(boff, _BLK)], dv, isem
            ).wait()

        def g_start(src_v, j, q):
            pltpu.async_copy(tab_hbm.at[src_v.at[j]], bufs[q], gsems[q])

        def g_wait(src_v, j, q):
            pltpu.make_async_copy(
                tab_hbm.at[src_v.at[j]], bufs[q], gsems[q]
            ).wait()

        def s_start(dst_v, j, q):
            pltpu.async_copy(
                bufs[q], accum.at[dst_v.at[j]], ssems[q], add=True
            )

        def s_wait(dst_v, j, q):
            pltpu.make_async_copy(
                bufs[q], accum.at[dst_v.at[j]], ssems[q]
            ).wait()

        def fill_zero(i, _):
            for kk in range(_CW // 16):
                zbuf[i, pl.ds(kk * 16, 16)] = jnp.zeros((16,), jnp.float32)
            return ()

        lax.fori_loop(0, 16, fill_zero, ())

        for p in range(2):
            tbl = 2 * p + c
            idx_start(0, 0, tbl)
            _zero_slices(accum, zbuf, s)
            plsc.subcore_barrier()

            for bi in range(nblk):
                par = bi % 2
                idx_wait(bi, par, tbl)
                if bi + 1 < nblk:
                    idx_start(bi + 1, (bi + 1) % 2, tbl)
                src_v, dst_v = idx_refs(par)

                g_start(src_v, 0, 0)
                g_start(src_v, 1, 1)
                g_wait(src_v, 0, 0)
                s_start(dst_v, 0, 0)
                g_start(src_v, 2, 2)
                g_wait(src_v, 1, 1)
                s_start(dst_v, 1, 1)
                s_wait(dst_v, 0, 0)
                g_start(src_v, 3, 0)
                last = _BLK - 1
                ntrip = (_BLK - 2) // 3

                def body(t, _):
                    j2 = 3 * t + 2
                    g_wait(src_v, j2, 2)
                    s_start(dst_v, j2, 2)
                    s_wait(dst_v, j2 - 1, 1)
                    g_start(src_v, j2 + 2, 1)
                    g_wait(src_v, j2 + 1, 0)
                    s_start(dst_v, j2 + 1, 0)

                    @pl.when(j2 + 3 <= last)
                    def _():
                        s_wait(dst_v, j2, 2)
                        g_start(src_v, j2 + 3, 2)

                    g_wait(src_v, j2 + 2, 1)
                    s_start(dst_v, j2 + 2, 1)

                    @pl.when(j2 + 4 <= last)
                    def _():
                        s_wait(dst_v, j2 + 1, 0)
                        g_start(src_v, j2 + 4, 0)

                    return ()

                lax.fori_loop(0, ntrip, body, ())
                s_wait(dst_v, last - 2, 2)
                s_wait(dst_v, last - 1, 0)
                s_wait(dst_v, last, 1)

            plsc.subcore_barrier()
            _writeout(accum, out_hbm.at[tbl], s)
            if p == 0:
                plsc.subcore_barrier()

    return k(tab, srcg, dst2d)


def _wprep_call(w0, w1):
    def k(w0_ref, w1_ref, out_ref):
        out_ref[:, :] = jnp.dot(
            w0_ref[:, :], w1_ref[:, :], preferred_element_type=jnp.float32
        )

    return pl.pallas_call(
        k, out_shape=jax.ShapeDtypeStruct((w0.shape[0], _D), jnp.float32)
    )(w0, w1)


def _dinv_of(degp_ref):
    deg = degp_ref[0, :, 0:1] + degp_ref[1, :, 0:1] + 1.0
    return lax.rsqrt(deg)


def _m1_call(x, w01, degp):
    n, kdim = x.shape

    def k(x_ref, w_ref, degp_ref, out_ref):
        dinv = _dinv_of(degp_ref)
        g = jnp.dot(x_ref[:, :], w_ref[:, :], preferred_element_type=jnp.float32)
        gs = g * dinv
        for cc in range(_NT):
            out_ref[cc, :, :] = gs[:, cc * _CW:(cc + 1) * _CW]

    return pl.pallas_call(
        k,
        grid=(n // _BN,),
        in_specs=[
            pl.BlockSpec((_BN, kdim), lambda i: (i, 0)),
            pl.BlockSpec((kdim, _D), lambda i: (0, 0)),
            pl.BlockSpec((2, _BN, 128), lambda i: (0, i, 0)),
        ],
        out_specs=pl.BlockSpec((_NT, _BN, _CW), lambda i: (0, i, 0)),
        out_shape=jax.ShapeDtypeStruct((_NT, n, _CW), jnp.float32),
    )(x, w01, degp)


def _mmid_call(acc, gs, degp, b4, w4):
    def k(acc_ref, gs_ref, degp_ref, b_ref, w_ref, out_ref):
        dinv = _dinv_of(degp_ref)
        o = jnp.zeros((_BN, _D), jnp.float32)
        for cc in range(_NT):
            a_c = jnp.maximum(
                dinv * (acc_ref[cc] + gs_ref[cc]) + b_ref[cc, :][None, :], 0.0
            )
            o = o + jnp.dot(a_c, w_ref[cc], preferred_element_type=jnp.float32)
        o = o * dinv
        for cc in range(_NT):
            out_ref[cc, :, :] = o[:, cc * _CW:(cc + 1) * _CW]

    return pl.pallas_call(
        k,
        grid=(_N // _BN,),
        in_specs=[
            pl.BlockSpec((_NT, _BN, _CW), lambda i: (0, i, 0)),
            pl.BlockSpec((_NT, _BN, _CW), lambda i: (0, i, 0)),
            pl.BlockSpec((2, _BN, 128), lambda i: (0, i, 0)),
            pl.BlockSpec((_NT, _CW), lambda i: (0, 0)),
            pl.BlockSpec((_NT, _CW, _D), lambda i: (0, 0, 0)),
        ],
        out_specs=pl.BlockSpec((_NT, _BN, _CW), lambda i: (0, i, 0)),
        out_shape=jax.ShapeDtypeStruct((_NT, _N, _CW), jnp.float32),
    )(acc, gs, degp, b4, w4)


def _final_call(acc, gs, degp, b4):
    def k(acc_ref, gs_ref, degp_ref, b_ref, out_ref):
        dinv = _dinv_of(degp_ref)
        for cc in range(_NT):
            out_ref[:, cc * _CW:(cc + 1) * _CW] = jnp.maximum(
                dinv * (acc_ref[cc] + gs_ref[cc]) + b_ref[cc, :][None, :], 0.0
            )

    return pl.pallas_call(
        k,
        grid=(_N // _BN,),
        in_specs=[
            pl.BlockSpec((_NT, _BN, _CW), lambda i: (0, i, 0)),
            pl.BlockSpec((_NT, _BN, _CW), lambda i: (0, i, 0)),
            pl.BlockSpec((2, _BN, 128), lambda i: (0, i, 0)),
            pl.BlockSpec((_NT, _CW), lambda i: (0, 0)),
        ],
        out_specs=pl.BlockSpec((_BN, _D), lambda i: (i, 0)),
        out_shape=jax.ShapeDtypeStruct((_N, _D), jnp.float32),
    )(acc, gs, degp, b4)


def kernel(x, edge_index, W0, W1, b1, W2, b2, W3, b3):
    e = edge_index.shape[1]
    src = edge_index[0].astype(jnp.int32)
    dst = edge_index[1].astype(jnp.int32)

    rows = -(-e // (512 * 64)) * 512
    npad = rows * 64 - e
    pad_ar = jnp.arange(npad, dtype=jnp.int32)
    src_p = jnp.concatenate([src, pad_ar % _N]).reshape(rows, 64)
    dst_p = jnp.concatenate([dst, _N + pad_ar % _PAD_ROWS]).reshape(rows, 64)
    srcg = src_p[None, :, :] + (_N * jnp.arange(_NT, dtype=jnp.int32))[:, None, None]

    degp = _deg_call(dst_p)
    w01 = _wprep_call(W0, W1)
    b1_4 = b1.reshape(_NT, _CW)
    b2_4 = b2.reshape(_NT, _CW)
    b3_4 = b3.reshape(_NT, _CW)
    w2_4 = W2.reshape(_NT, _CW, _D)
    w3_4 = W3.reshape(_NT, _CW, _D)

    gs1 = _m1_call(x, w01, degp)
    acc1 = _conv_call(gs1.reshape(_NT * _N, _CW), srcg, dst_p)
    gs2 = _mmid_call(acc1, gs1, degp, b1_4, w2_4)
    acc2 = _conv_call(gs2.reshape(_NT * _N, _CW), srcg, dst_p)
    gs3 = _mmid_call(acc2, gs2, degp, b2_4, w3_4)
    acc3 = _conv_call(gs3.reshape(_NT * _N, _CW), srcg, dst_p)
    h = _final_call(acc3, gs3, degp, b3_4)
    return h.reshape(100, -1, _D)
